# Initial kernel scaffold; baseline (speedup 1.0000x reference)
import jax
import jax.numpy as jnp
from jax.experimental import pallas as pl


def kernel(x_u, x_v, sup_vals, weights_u, weights_v, sup_rows, sup_cols):
    raise NotImplementedError("write your pallas kernel here")


# speedup vs baseline: 2.1294x; 2.1294x over previous
import functools

import jax
import jax.numpy as jnp
from jax import lax
from jax.experimental import pallas as pl
from jax.experimental.pallas import tpu as pltpu
from jax.experimental.pallas import tpu_sc as plsc


def _y_body(w_ref, x_ref, y_ref, wacc):
    i = pl.program_id(0)
    r = pl.program_id(1)

    @pl.when(r == 0)
    def _():
        prev = jnp.where(i == 0, jnp.zeros_like(wacc[...]), wacc[...])
        wacc[...] = prev + w_ref[0]

    y_ref[0] = jnp.dot(x_ref[...], wacc[...], preferred_element_type=jnp.float32)


def _compute_y(x, weights):
    s, d_in, d_out = weights.shape
    n = x.shape[0]
    br = 1000
    return pl.pallas_call(
        _y_body,
        grid=(s, n // br),
        in_specs=[
            pl.BlockSpec((1, d_in, d_out), lambda i, r: (i, 0, 0)),
            pl.BlockSpec((br, d_in), lambda i, r: (r, 0)),
        ],
        out_specs=pl.BlockSpec((1, br, d_out), lambda i, r: (i, r, 0)),
        out_shape=jax.ShapeDtypeStruct((s, n, d_out), jnp.float32),
        scratch_shapes=[pltpu.VMEM((d_in, d_out), jnp.float32)],
    )(weights, x)


_LANES = 16
_HALF = 128
_CH = 80
_WB = 80


def _sc_agg(y_flat, gidx, sidx, vals, n_dst, n_src):
    s_sup, e_edges = gidx.shape
    gidx = gidx.reshape(-1)
    sidx = sidx.reshape(-1)
    vals = vals.reshape(-1)
    n_tiles = 16
    epc = e_edges // n_tiles
    n_chunks = epc // _CH
    row_chunks = n_dst // _WB
    wb_iters = -(-row_chunks // n_tiles)

    mesh = plsc.VectorSubcoreMesh(core_axis_name="c", subcore_axis_name="s")

    @functools.partial(
        pl.kernel,
        mesh=mesh,
        out_type=jax.ShapeDtypeStruct((n_dst, 2 * _HALF), jnp.float32),
        scratch_types=[
            pltpu.VMEM((_CH,), jnp.int32),
            pltpu.VMEM((_CH,), jnp.int32),
            pltpu.VMEM((_CH,), jnp.float32),
            pltpu.VMEM((_CH, _HALF), jnp.float32),
            pltpu.VMEM((_WB, _HALF), jnp.float32),
            pltpu.VMEM_SHARED((n_dst, _HALF), jnp.float32),
            pltpu.SemaphoreType.DMA,
        ],
    )
    def agg(y_hbm, g_hbm, s_hbm, v_hbm, out_hbm, gbuf, sbuf, vbuf, rbuf, wbuf,
            acc, sem):
        c = lax.axis_index("c")
        s = lax.axis_index("s")

        def zero_body(e, carry):
            for q in range(_HALF // _LANES):
                wbuf[e, pl.ds(q * _LANES, _LANES)] = jnp.zeros(
                    (_LANES,), jnp.float32)
            return carry

        lax.fori_loop(0, _WB, zero_body, 0)

        def zero_chunk(j, carry):
            idx = s + j * n_tiles

            @pl.when(idx < row_chunks)
            def _():
                pltpu.sync_copy(wbuf, acc.at[pl.ds(idx * _WB, _WB)])

            return carry

        lax.fori_loop(0, wb_iters, zero_chunk, 0)
        plsc.subcore_barrier()

        for i in range(s_sup):
            base = jnp.int32(i * 2 * n_src) + c

            def chunk_body(k, carry, i=i, base=base):
                e0 = i * e_edges + s * epc + k * _CH
                pltpu.sync_copy(g_hbm.at[pl.ds(e0, _CH)], gbuf)
                pltpu.sync_copy(s_hbm.at[pl.ds(e0, _CH)], sbuf)
                pltpu.sync_copy(v_hbm.at[pl.ds(e0, _CH)], vbuf)

                def gfix(m, carry2):
                    sl = gbuf[pl.ds(m * _LANES, _LANES)]
                    gbuf[pl.ds(m * _LANES, _LANES)] = sl * 2 + base
                    return carry2

                lax.fori_loop(0, _CH // _LANES, gfix, 0)
                pltpu.async_copy(y_hbm.at[gbuf], rbuf, sem).wait()

                def scale(g, carry2):
                    vv = vbuf[pl.ds(g * _LANES, _LANES)]
                    for j in range(_LANES):
                        e = g * _LANES + j
                        v = vv[j]
                        for q in range(_HALF // _LANES):
                            sl = pl.ds(q * _LANES, _LANES)
                            rbuf[e, sl] = rbuf[e, sl] * v
                    return carry2

                lax.fori_loop(0, _CH // _LANES, scale, 0)
                pltpu.sync_copy(rbuf, acc.at[sbuf], add=True)
                return carry

            lax.fori_loop(0, n_chunks, chunk_body, 0)

        plsc.subcore_barrier()

        def wb_chunk(j, carry):
            idx = s + j * n_tiles

            @pl.when(idx < row_chunks)
            def _():
                r0 = idx * _WB
                pltpu.sync_copy(acc.at[pl.ds(r0, _WB)], wbuf)

                def relu_body(e, carry2):
                    for q in range(_HALF // _LANES):
                        sl = pl.ds(q * _LANES, _LANES)
                        wbuf[e, sl] = jnp.maximum(wbuf[e, sl], 0.0)
                    return carry2

                lax.fori_loop(0, _WB, relu_body, 0)
                pltpu.sync_copy(wbuf, out_hbm.at[pl.ds(r0, _WB),
                                                 pl.ds(c * _HALF, _HALF)])

            return carry

        lax.fori_loop(0, wb_iters, wb_chunk, 0)

    return agg(y_flat, gidx, sidx, vals)


def kernel(x_u, x_v, sup_vals, weights_u, weights_v, sup_rows, sup_cols):
    n_u = x_u.shape[0]
    n_v = x_v.shape[0]
    rows = sup_rows.astype(jnp.int32)
    cols = sup_cols.astype(jnp.int32)
    vals = sup_vals.astype(jnp.float32)

    y_v = _compute_y(x_v, weights_v)
    y_u = _compute_y(x_u, weights_u)
    y_v2 = y_v.reshape(-1, _HALF)
    y_u2 = y_u.reshape(-1, _HALF)

    z_u = _sc_agg(y_v2, cols, rows, vals, n_u, n_v)
    z_v = _sc_agg(y_u2, rows, cols, vals, n_v, n_u)
    return z_u, z_v


# speedup vs baseline: 4.3790x; 2.0565x over previous
import functools

import jax
import jax.numpy as jnp
from jax import lax
from jax.experimental import pallas as pl
from jax.experimental.pallas import tpu as pltpu
from jax.experimental.pallas import tpu_sc as plsc


def _y_body(w_ref, x_ref, y_ref, wacc):
    i = pl.program_id(0)
    r = pl.program_id(1)

    @pl.when(r == 0)
    def _():
        prev = jnp.where(i == 0, jnp.zeros_like(wacc[...]), wacc[...])
        wacc[...] = prev + w_ref[0]

    y_ref[0] = jnp.dot(x_ref[...], wacc[...], preferred_element_type=jnp.float32)


def _compute_y(x, weights):
    s, d_in, d_out = weights.shape
    n = x.shape[0]
    br = 1000
    return pl.pallas_call(
        _y_body,
        grid=(s, n // br),
        in_specs=[
            pl.BlockSpec((1, d_in, d_out), lambda i, r: (i, 0, 0)),
            pl.BlockSpec((br, d_in), lambda i, r: (r, 0)),
        ],
        out_specs=pl.BlockSpec((1, br, d_out), lambda i, r: (i, r, 0)),
        out_shape=jax.ShapeDtypeStruct((s, n, d_out), jnp.float32),
        scratch_shapes=[pltpu.VMEM((d_in, d_out), jnp.float32)],
    )(weights, x)


_LANES = 16
_HALF = 128
_CH = 80
_WB = 80


def _sc_agg(y_flat, gidx, sidx, vals, n_dst, n_src):
    s_sup, e_edges = gidx.shape
    n_tiles = 16
    epc = e_edges // n_tiles
    n_chunks = epc // _CH
    gidx = gidx.reshape(s_sup, n_tiles, n_chunks, _CH)
    sidx = sidx.reshape(s_sup, n_tiles, n_chunks, _CH)
    vals = vals.reshape(s_sup, n_tiles, n_chunks, _CH)
    row_chunks = n_dst // _WB
    wb_iters = -(-row_chunks // n_tiles)

    mesh = plsc.VectorSubcoreMesh(core_axis_name="c", subcore_axis_name="s")

    @functools.partial(
        pl.kernel,
        mesh=mesh,
        out_type=jax.ShapeDtypeStruct((n_dst, 2 * _HALF), jnp.float32),
        scratch_types=[
            pltpu.VMEM((n_chunks, _CH), jnp.int32),
            pltpu.VMEM((n_chunks, _CH), jnp.int32),
            pltpu.VMEM((n_chunks, _CH), jnp.float32),
            pltpu.VMEM((_CH, _HALF), jnp.float32),
            pltpu.VMEM((_CH, _HALF), jnp.float32),
            pltpu.VMEM((_WB, _HALF), jnp.float32),
            pltpu.VMEM_SHARED((n_dst, _HALF), jnp.float32),
            pltpu.SemaphoreType.DMA,
            pltpu.SemaphoreType.DMA,
        ],
    )
    def agg(y_hbm, g_hbm, s_hbm, v_hbm, out_hbm, gbuf, sbuf, vbuf, rb0, rb1,
            wbuf, acc, sem0, sem1):
        c = lax.axis_index("c")
        s = lax.axis_index("s")

        def zero_body(e, carry):
            for q in range(_HALF // _LANES):
                wbuf[e, pl.ds(q * _LANES, _LANES)] = jnp.zeros(
                    (_LANES,), jnp.float32)
            return carry

        lax.fori_loop(0, _WB, zero_body, 0)

        def zero_chunk(j, carry):
            idx = s + j * n_tiles

            @pl.when(idx < row_chunks)
            def _():
                pltpu.sync_copy(wbuf, acc.at[pl.ds(idx * _WB, _WB)])

            return carry

        lax.fori_loop(0, wb_iters, zero_chunk, 0)
        plsc.subcore_barrier()

        def start_gather(k, rb, sem):
            return pltpu.async_copy(y_hbm.at[gbuf.at[k]], rb, sem)

        def wait_gather(rb, sem):
            pltpu.make_async_copy(y_hbm.at[pl.ds(0, _CH)], rb, sem).wait()

        def process(k, rb):
            def scale(g, carry2):
                vv = vbuf[k, pl.ds(g * _LANES, _LANES)]
                for j in range(_LANES):
                    e = g * _LANES + j
                    v = vv[j]
                    for q in range(_HALF // _LANES):
                        sl = pl.ds(q * _LANES, _LANES)
                        rb[e, sl] = rb[e, sl] * v
                return carry2

            lax.fori_loop(0, _CH // _LANES, scale, 0)
            pltpu.sync_copy(rb, acc.at[sbuf.at[k]], add=True)

        for i in range(s_sup):
            base = jnp.int32(i * 2 * n_src) + c

            pltpu.sync_copy(g_hbm.at[i, s], gbuf)
            pltpu.sync_copy(s_hbm.at[i, s], sbuf)
            pltpu.sync_copy(v_hbm.at[i, s], vbuf)

            def gfix(k, carry, base=base):
                for m in range(_CH // _LANES):
                    sl = pl.ds(m * _LANES, _LANES)
                    gbuf[k, sl] = gbuf[k, sl] * 2 + base
                return carry

            lax.fori_loop(0, n_chunks, gfix, 0)

            start_gather(0, rb0, sem0)

            def pair(p, carry):
                k0 = 2 * p
                k1 = 2 * p + 1

                @pl.when(k1 < n_chunks)
                def _():
                    start_gather(k1, rb1, sem1)

                wait_gather(rb0, sem0)
                process(k0, rb0)

                @pl.when(k1 < n_chunks)
                def _():
                    @pl.when(k1 + 1 < n_chunks)
                    def _():
                        start_gather(k1 + 1, rb0, sem0)

                    wait_gather(rb1, sem1)
                    process(k1, rb1)

                return carry

            lax.fori_loop(0, -(-n_chunks // 2), pair, 0)

        plsc.subcore_barrier()

        def wb_chunk(j, carry):
            idx = s + j * n_tiles

            @pl.when(idx < row_chunks)
            def _():
                r0 = idx * _WB
                pltpu.sync_copy(acc.at[pl.ds(r0, _WB)], wbuf)

                def relu_body(e, carry2):
                    for q in range(_HALF // _LANES):
                        sl = pl.ds(q * _LANES, _LANES)
                        wbuf[e, sl] = jnp.maximum(wbuf[e, sl], 0.0)
                    return carry2

                lax.fori_loop(0, _WB, relu_body, 0)
                pltpu.sync_copy(wbuf, out_hbm.at[pl.ds(r0, _WB),
                                                 pl.ds(c * _HALF, _HALF)])

            return carry

        lax.fori_loop(0, wb_iters, wb_chunk, 0)

    return agg(y_flat, gidx, sidx, vals)


def kernel(x_u, x_v, sup_vals, weights_u, weights_v, sup_rows, sup_cols):
    n_u = x_u.shape[0]
    n_v = x_v.shape[0]
    rows = sup_rows.astype(jnp.int32)
    cols = sup_cols.astype(jnp.int32)
    vals = sup_vals.astype(jnp.float32)

    y_v = _compute_y(x_v, weights_v)
    y_u = _compute_y(x_u, weights_u)
    y_v2 = y_v.reshape(-1, _HALF)
    y_u2 = y_u.reshape(-1, _HALF)

    z_u = _sc_agg(y_v2, cols, rows, vals, n_u, n_v)
    z_v = _sc_agg(y_u2, rows, cols, vals, n_v, n_u)
    return z_u, z_v
